# Initial kernel scaffold; baseline (speedup 1.0000x reference)
import functools

import jax
import jax.numpy as jnp
from jax import lax
from jax.experimental import pallas as pl
from jax.experimental.pallas import tpu as pltpu
from jax.experimental.pallas import tpu_sc as plsc

N_NODES = 10000
D = 128
E = 320000

NC, NS = 2, 16
NW = NC * NS
K = 128
CPW = 80
E_PAD = NW * CPW * K
N_PAD = 10016
RPT = N_PAD // NS
DW = 16

_mesh = plsc.VectorSubcoreMesh(core_axis_name="c", subcore_axis_name="s")


@functools.partial(
    pl.kernel,
    out_type=jax.ShapeDtypeStruct((NC, N_PAD, DW), jnp.float32),
    mesh=_mesh,
    scratch_types=[
        pltpu.VMEM((K,), jnp.int32),
        pltpu.VMEM((K, DW), jnp.float32),
        pltpu.VMEM_SHARED((N_PAD, DW), jnp.float32),
    ],
)
def _deg_sc(dst_hbm, ones_hbm, zeros_hbm, deg_out, idx_v, ones_v, acc_sh):
    c = lax.axis_index("c")
    s = lax.axis_index("s")
    wid = s * NC + c
    r0 = s * RPT
    pltpu.sync_copy(zeros_hbm.at[pl.ds(r0, RPT)], acc_sh.at[pl.ds(r0, RPT)])
    pltpu.sync_copy(ones_hbm, ones_v)
    plsc.subcore_barrier()

    def body(i, carry):
        base = (wid * CPW + i) * K
        pltpu.sync_copy(dst_hbm.at[pl.ds(base, K)], idx_v)
        pltpu.sync_copy(ones_v, acc_sh.at[idx_v], add=True)
        return carry

    lax.fori_loop(0, CPW, body, 0)
    plsc.subcore_barrier()
    pltpu.sync_copy(acc_sh.at[pl.ds(r0, RPT)], deg_out.at[c, pl.ds(r0, RPT)])


@functools.partial(
    pl.kernel,
    out_type=jax.ShapeDtypeStruct((NC, N_PAD, D), jnp.float32),
    mesh=_mesh,
    scratch_types=[
        pltpu.VMEM((K,), jnp.int32),
        pltpu.VMEM((K,), jnp.int32),
        pltpu.VMEM((K, D), jnp.float32),
        pltpu.VMEM_SHARED((N_PAD, D), jnp.float32),
        pltpu.SemaphoreType.DMA,
    ],
)
def _scatter_sc(y_hbm, src_hbm, dst_hbm, zeros_hbm, acc_out,
                sidx, didx, rows, acc_sh, sem):
    c = lax.axis_index("c")
    s = lax.axis_index("s")
    wid = s * NC + c
    r0 = s * RPT
    pltpu.sync_copy(zeros_hbm.at[pl.ds(r0, RPT)], acc_sh.at[pl.ds(r0, RPT)])
    plsc.subcore_barrier()

    def body(i, carry):
        base = (wid * CPW + i) * K
        pltpu.sync_copy(src_hbm.at[pl.ds(base, K)], sidx)
        pltpu.sync_copy(dst_hbm.at[pl.ds(base, K)], didx)
        pltpu.async_copy(y_hbm.at[sidx], rows, sem).wait()
        pltpu.sync_copy(rows, acc_sh.at[didx], add=True)
        return carry

    lax.fori_loop(0, CPW, body, 0)
    plsc.subcore_barrier()
    pltpu.sync_copy(acc_sh.at[pl.ds(r0, RPT)], acc_out.at[c, pl.ds(r0, RPT)])


def _dis(p0, p1):
    return lax.rsqrt(p0[:, 0:1] + p1[:, 0:1] + 1.0)


def _dense1_body(p0, p1, x_ref, w_ref, y_ref):
    xw = jnp.dot(x_ref[...], w_ref[...], preferred_element_type=jnp.float32)
    y_ref[...] = _dis(p0, p1) * xw


def _dense2_body(p0, p1, a0, a1, y1, w_ref, b_ref, y2_ref):
    dis = _dis(p0, p1)
    h = jnp.maximum(dis * (a0[...] + a1[...] + y1[...]) + b_ref[...], 0.0)
    y2_ref[...] = dis * jnp.dot(h, w_ref[...], preferred_element_type=jnp.float32)


def _dense3_body(p0, p1, a0, a1, y2, b_ref, o_ref):
    o_ref[...] = _dis(p0, p1) * (a0[...] + a1[...] + y2[...]) + b_ref[...]


_dense1 = pl.pallas_call(
    _dense1_body, out_shape=jax.ShapeDtypeStruct((N_PAD, D), jnp.float32))
_dense2 = pl.pallas_call(
    _dense2_body, out_shape=jax.ShapeDtypeStruct((N_PAD, D), jnp.float32))
_dense3 = pl.pallas_call(
    _dense3_body, out_shape=jax.ShapeDtypeStruct((N_PAD, D), jnp.float32))


def kernel(x, edge_index, W1, b1, W2, b2):
    ei = edge_index.astype(jnp.int32)
    pad = E_PAD - E
    src = jnp.concatenate([ei[0], jnp.zeros((pad,), jnp.int32)])
    dst = jnp.concatenate([ei[1], jnp.full((pad,), N_NODES, jnp.int32)])
    x_pad = jnp.zeros((N_PAD, D), jnp.float32).at[:N_NODES].set(x)
    zeros_nd = jnp.zeros((N_PAD, D), jnp.float32)
    zeros_dw = jnp.zeros((N_PAD, DW), jnp.float32)
    ones_k = jnp.ones((K, DW), jnp.float32)

    degp = _deg_sc(dst, ones_k, zeros_dw)
    p0, p1 = degp[0], degp[1]

    y1 = _dense1(p0, p1, x_pad, W1)
    acc1 = _scatter_sc(y1, src, dst, zeros_nd)
    y2 = _dense2(p0, p1, acc1[0], acc1[1], y1, W2, b1.reshape(1, D))
    acc2 = _scatter_sc(y2, src, dst, zeros_nd)
    out = _dense3(p0, p1, acc2[0], acc2[1], y2, b2.reshape(1, D))
    return out[:N_NODES]


# speedup vs baseline: 7.8638x; 7.8638x over previous
import functools

import jax
import jax.numpy as jnp
from jax import lax
from jax.experimental import pallas as pl
from jax.experimental.pallas import tpu as pltpu
from jax.experimental.pallas import tpu_sc as plsc

N_NODES = 10000
D = 128
E = 320000

NC, NS = 2, 16
NW = NC * NS
K = 128
CPW = 80
E_PAD = NW * CPW * K
N_PAD = 10112
RPT = N_PAD // NS
DW = 16

_mesh = plsc.VectorSubcoreMesh(core_axis_name="c", subcore_axis_name="s")


@functools.partial(
    pl.kernel,
    out_type=jax.ShapeDtypeStruct((NC, N_PAD, DW), jnp.float32),
    mesh=_mesh,
    scratch_types=[
        pltpu.VMEM((K,), jnp.int32),
        pltpu.VMEM((K, DW), jnp.float32),
        pltpu.VMEM_SHARED((N_PAD, DW), jnp.float32),
    ],
)
def _deg_sc(dst_hbm, ones_hbm, zeros_hbm, deg_out, idx_v, ones_v, acc_sh):
    c = lax.axis_index("c")
    s = lax.axis_index("s")
    wid = s * NC + c
    r0 = s * RPT
    pltpu.sync_copy(zeros_hbm.at[pl.ds(r0, RPT)], acc_sh.at[pl.ds(r0, RPT)])
    pltpu.sync_copy(ones_hbm, ones_v)
    plsc.subcore_barrier()

    def body(i, carry):
        base = (wid * CPW + i) * K
        pltpu.sync_copy(dst_hbm.at[pl.ds(base, K)], idx_v)
        pltpu.sync_copy(ones_v, acc_sh.at[idx_v], add=True)
        return carry

    lax.fori_loop(0, CPW, body, 0)
    plsc.subcore_barrier()
    pltpu.sync_copy(acc_sh.at[pl.ds(r0, RPT)], deg_out.at[c, pl.ds(r0, RPT)])


@functools.partial(
    pl.kernel,
    out_type=jax.ShapeDtypeStruct((NC, N_PAD, D), jnp.float32),
    mesh=_mesh,
    scratch_types=[
        pltpu.VMEM((K,), jnp.int32),
        pltpu.VMEM((K,), jnp.int32),
        pltpu.VMEM((K, D), jnp.float32),
        pltpu.VMEM_SHARED((N_PAD, D), jnp.float32),
        pltpu.SemaphoreType.DMA,
    ],
)
def _scatter_sc(y_hbm, src_hbm, dst_hbm, zeros_hbm, acc_out,
                sidx, didx, rows, acc_sh, sem):
    c = lax.axis_index("c")
    s = lax.axis_index("s")
    wid = s * NC + c
    r0 = s * RPT
    pltpu.sync_copy(zeros_hbm.at[pl.ds(r0, RPT)], acc_sh.at[pl.ds(r0, RPT)])
    plsc.subcore_barrier()

    def body(i, carry):
        base = (wid * CPW + i) * K
        pltpu.sync_copy(src_hbm.at[pl.ds(base, K)], sidx)
        pltpu.sync_copy(dst_hbm.at[pl.ds(base, K)], didx)
        pltpu.async_copy(y_hbm.at[sidx], rows, sem).wait()
        pltpu.sync_copy(rows, acc_sh.at[didx], add=True)
        return carry

    lax.fori_loop(0, CPW, body, 0)
    plsc.subcore_barrier()
    pltpu.sync_copy(acc_sh.at[pl.ds(r0, RPT)], acc_out.at[c, pl.ds(r0, RPT)])


def _dis(p0, p1):
    return lax.rsqrt(p0[:, 0:1] + p1[:, 0:1] + 1.0)


def _dense1_body(p0, p1, x_ref, w_ref, y_ref):
    xw = jnp.dot(x_ref[...], w_ref[...], preferred_element_type=jnp.float32)
    y_ref[...] = _dis(p0, p1) * xw


def _dense2_body(p0, p1, a0, a1, y1, w_ref, b_ref, y2_ref):
    dis = _dis(p0, p1)
    h = jnp.maximum(dis * (a0[...] + a1[...] + y1[...]) + b_ref[...], 0.0)
    y2_ref[...] = dis * jnp.dot(h, w_ref[...], preferred_element_type=jnp.float32)


def _dense3_body(p0, p1, a0, a1, y2, b_ref, o_ref):
    o_ref[...] = _dis(p0, p1) * (a0[...] + a1[...] + y2[...]) + b_ref[...]


_dense1 = pl.pallas_call(
    _dense1_body, out_shape=jax.ShapeDtypeStruct((N_PAD, D), jnp.float32))
_dense2 = pl.pallas_call(
    _dense2_body, out_shape=jax.ShapeDtypeStruct((N_PAD, D), jnp.float32))
_dense3 = pl.pallas_call(
    _dense3_body, out_shape=jax.ShapeDtypeStruct((N_PAD, D), jnp.float32))


def kernel(x, edge_index, W1, b1, W2, b2):
    ei = edge_index.astype(jnp.int32)
    pad = E_PAD - E
    src = jnp.concatenate([ei[0], jnp.zeros((pad,), jnp.int32)])
    dst = jnp.concatenate([ei[1], jnp.full((pad,), N_NODES, jnp.int32)])
    x_pad = jnp.zeros((N_PAD, D), jnp.float32).at[:N_NODES].set(x)
    zeros_nd = jnp.zeros((N_PAD, D), jnp.float32)
    zeros_dw = jnp.zeros((N_PAD, DW), jnp.float32)
    ones_k = jnp.ones((K, DW), jnp.float32)

    degp = _deg_sc(dst, ones_k, zeros_dw)
    p0, p1 = degp[0], degp[1]

    y1 = _dense1(p0, p1, x_pad, W1)
    acc1 = _scatter_sc(y1, src, dst, zeros_nd)
    y2 = _dense2(p0, p1, acc1[0], acc1[1], y1, W2, b1.reshape(1, D))
    acc2 = _scatter_sc(y2, src, dst, zeros_nd)
    out = _dense3(p0, p1, acc2[0], acc2[1], y2, b2.reshape(1, D))
    return out[:N_NODES]


# speedup vs baseline: 8.7177x; 1.1086x over previous
import functools

import jax
import jax.numpy as jnp
from jax import lax
from jax.experimental import pallas as pl
from jax.experimental.pallas import tpu as pltpu
from jax.experimental.pallas import tpu_sc as plsc

N_NODES = 10000
D = 128
E = 320000

NC, NS = 2, 16
NW = NC * NS
K = 128
CPW = 80
HALF = CPW // 2
E_PAD = NW * CPW * K
N_PAD = 10112
RPT = N_PAD // NS
DW = 16

_mesh = plsc.VectorSubcoreMesh(core_axis_name="c", subcore_axis_name="s")


NB = 2
DEG_Q = 10


@functools.partial(
    pl.kernel,
    out_type=jax.ShapeDtypeStruct((NC, N_PAD, DW), jnp.float32),
    mesh=_mesh,
    scratch_types=[
        pltpu.VMEM((K,), jnp.int32),
        pltpu.VMEM((K, DW), jnp.float32),
        pltpu.VMEM_SHARED((N_PAD, DW), jnp.float32),
    ],
)
def _deg_sc(dst_hbm, ones_hbm, zeros_hbm, deg_out, idx_v, ones_v, acc_sh):
    c = lax.axis_index("c")
    s = lax.axis_index("s")
    wid = s * NC + c
    r0 = s * RPT
    pltpu.sync_copy(zeros_hbm.at[pl.ds(r0, RPT)], acc_sh.at[pl.ds(r0, RPT)])
    pltpu.sync_copy(ones_hbm, ones_v)
    plsc.subcore_barrier()

    def body(i, carry):
        pltpu.sync_copy(dst_hbm.at[wid, i], idx_v)
        pltpu.sync_copy(ones_v, acc_sh.at[idx_v], add=True)
        return carry

    lax.fori_loop(0, CPW, body, 0)
    plsc.subcore_barrier()
    pltpu.sync_copy(acc_sh.at[pl.ds(r0, RPT)], deg_out.at[c, pl.ds(r0, RPT)])


@functools.partial(
    pl.kernel,
    out_type=jax.ShapeDtypeStruct((NC, N_PAD, D), jnp.float32),
    mesh=_mesh,
    scratch_types=[
        pltpu.VMEM((HALF, K), jnp.int32),
        pltpu.VMEM((HALF, K), jnp.int32),
        [pltpu.VMEM((K, D), jnp.float32) for _ in range(NB)],
        pltpu.VMEM_SHARED((N_PAD, D), jnp.float32),
        [pltpu.SemaphoreType.DMA for _ in range(NB)],
    ],
)
def _scatter_sc(y_hbm, src_hbm, dst_hbm, zeros_hbm, acc_out,
                sidx_all, didx_all, rows, acc_sh, sems):
    c = lax.axis_index("c")
    s = lax.axis_index("s")
    wid = s * NC + c
    r0 = s * RPT
    pltpu.sync_copy(zeros_hbm.at[pl.ds(r0, RPT)], acc_sh.at[pl.ds(r0, RPT)])
    plsc.subcore_barrier()

    for h in range(2):
        pltpu.sync_copy(src_hbm.at[wid, pl.ds(h * HALF, HALF)], sidx_all)
        pltpu.sync_copy(dst_hbm.at[wid, pl.ds(h * HALF, HALF)], didx_all)

        def body(j, carry):
            pltpu.async_copy(y_hbm.at[sidx_all.at[j]], rows[0], sems[0]).wait()
            pltpu.sync_copy(rows[0], acc_sh.at[didx_all.at[j]], add=True)
            return carry

        lax.fori_loop(0, HALF, body, 0)
    plsc.subcore_barrier()
    pltpu.sync_copy(acc_sh.at[pl.ds(r0, RPT)], acc_out.at[c, pl.ds(r0, RPT)])


def _dis(p0, p1):
    return lax.rsqrt(p0[:, 0:1] + p1[:, 0:1] + 1.0)


def _dense1_body(p0, p1, x_ref, w_ref, y_ref):
    xw = jnp.dot(x_ref[...], w_ref[...], preferred_element_type=jnp.float32)
    y_ref[...] = _dis(p0, p1) * xw


def _dense2_body(p0, p1, a0, a1, y1, w_ref, b_ref, y2_ref):
    dis = _dis(p0, p1)
    h = jnp.maximum(dis * (a0[...] + a1[...] + y1[...]) + b_ref[...], 0.0)
    y2_ref[...] = dis * jnp.dot(h, w_ref[...], preferred_element_type=jnp.float32)


def _dense3_body(p0, p1, a0, a1, y2, b_ref, o_ref):
    o_ref[...] = _dis(p0, p1) * (a0[...] + a1[...] + y2[...]) + b_ref[...]


_dense1 = pl.pallas_call(
    _dense1_body, out_shape=jax.ShapeDtypeStruct((N_PAD, D), jnp.float32))
_dense2 = pl.pallas_call(
    _dense2_body, out_shape=jax.ShapeDtypeStruct((N_PAD, D), jnp.float32))
_dense3 = pl.pallas_call(
    _dense3_body, out_shape=jax.ShapeDtypeStruct((N_PAD, D), jnp.float32))


def kernel(x, edge_index, W1, b1, W2, b2):
    ei = edge_index.astype(jnp.int32)
    pad = E_PAD - E
    src = jnp.concatenate(
        [ei[0], jnp.zeros((pad,), jnp.int32)]).reshape(NW, CPW, K)
    dst = jnp.concatenate(
        [ei[1], jnp.full((pad,), N_NODES, jnp.int32)]).reshape(NW, CPW, K)
    x_pad = jnp.zeros((N_PAD, D), jnp.float32).at[:N_NODES].set(x)
    zeros_nd = jnp.zeros((N_PAD, D), jnp.float32)
    zeros_dw = jnp.zeros((N_PAD, DW), jnp.float32)
    ones_k = jnp.ones((K, DW), jnp.float32)

    degp = _deg_sc(dst, ones_k, zeros_dw)
    p0, p1 = degp[0], degp[1]

    y1 = _dense1(p0, p1, x_pad, W1)
    acc1 = _scatter_sc(y1, src, dst, zeros_nd)
    y2 = _dense2(p0, p1, acc1[0], acc1[1], y1, W2, b1.reshape(1, D))
    acc2 = _scatter_sc(y2, src, dst, zeros_nd)
    out = _dense3(p0, p1, acc2[0], acc2[1], y2, b2.reshape(1, D))
    return out[:N_NODES]


# speedup vs baseline: 8.7209x; 1.0004x over previous
import functools

import jax
import jax.numpy as jnp
from jax import lax
from jax.experimental import pallas as pl
from jax.experimental.pallas import tpu as pltpu
from jax.experimental.pallas import tpu_sc as plsc

N_NODES = 10000
D = 128
E = 320000

NC, NS = 2, 16
NW = NC * NS
K = 128
CPW = 80
HALF = CPW // 2
E_PAD = NW * CPW * K
N_PAD = 10112
RPT = N_PAD // NS
DW = 16

_mesh = plsc.VectorSubcoreMesh(core_axis_name="c", subcore_axis_name="s")


NB = 2
DEG_Q = 10


@functools.partial(
    pl.kernel,
    out_type=jax.ShapeDtypeStruct((NC, N_PAD, DW), jnp.float32),
    mesh=_mesh,
    scratch_types=[
        pltpu.VMEM((K,), jnp.int32),
        pltpu.VMEM((K, DW), jnp.float32),
        pltpu.VMEM_SHARED((N_PAD, DW), jnp.float32),
    ],
)
def _deg_sc(dst_hbm, ones_hbm, zeros_hbm, deg_out, idx_v, ones_v, acc_sh):
    c = lax.axis_index("c")
    s = lax.axis_index("s")
    wid = s * NC + c
    r0 = s * RPT
    pltpu.sync_copy(zeros_hbm.at[pl.ds(r0, RPT)], acc_sh.at[pl.ds(r0, RPT)])
    pltpu.sync_copy(ones_hbm, ones_v)
    plsc.subcore_barrier()

    def body(i, carry):
        pltpu.sync_copy(dst_hbm.at[wid, i], idx_v)
        pltpu.sync_copy(ones_v, acc_sh.at[idx_v], add=True)
        return carry

    lax.fori_loop(0, CPW, body, 0)
    plsc.subcore_barrier()
    pltpu.sync_copy(acc_sh.at[pl.ds(r0, RPT)], deg_out.at[c, pl.ds(r0, RPT)])


@functools.partial(
    pl.kernel,
    out_type=jax.ShapeDtypeStruct((NC, N_PAD, D), jnp.float32),
    mesh=_mesh,
    scratch_types=[
        pltpu.VMEM((HALF, K), jnp.int32),
        pltpu.VMEM((HALF, K), jnp.int32),
        [pltpu.VMEM((K, D), jnp.float32) for _ in range(NB)],
        pltpu.VMEM_SHARED((N_PAD, D), jnp.float32),
        pltpu.SemaphoreType.DMA,
        pltpu.SemaphoreType.DMA,
    ],
)
def _scatter_sc(y_hbm, src_hbm, dst_hbm, zeros_hbm, acc_out,
                sidx_all, didx_all, rows, acc_sh, sem_a, sem_b):
    sems = (sem_a, sem_b)
    c = lax.axis_index("c")
    s = lax.axis_index("s")
    wid = s * NC + c
    r0 = s * RPT
    pltpu.sync_copy(zeros_hbm.at[pl.ds(r0, RPT)], acc_sh.at[pl.ds(r0, RPT)])
    plsc.subcore_barrier()

    for h in range(2):
        pltpu.sync_copy(src_hbm.at[wid, pl.ds(h * HALF, HALF)], sidx_all)
        pltpu.sync_copy(dst_hbm.at[wid, pl.ds(h * HALF, HALF)], didx_all)

        def body(j, carry):
            pltpu.async_copy(y_hbm.at[sidx_all.at[j]], rows[0], sems[0]).wait()
            pltpu.sync_copy(rows[0], acc_sh.at[didx_all.at[j]], add=True)
            return carry

        lax.fori_loop(0, HALF, body, 0)
    plsc.subcore_barrier()
    pltpu.sync_copy(acc_sh.at[pl.ds(r0, RPT)], acc_out.at[c, pl.ds(r0, RPT)])


def _dis(p0, p1):
    return lax.rsqrt(p0[:, 0:1] + p1[:, 0:1] + 1.0)


def _dense1_body(p0, p1, x_ref, w_ref, y_ref):
    xw = jnp.dot(x_ref[...], w_ref[...], preferred_element_type=jnp.float32)
    y_ref[...] = _dis(p0, p1) * xw


def _dense2_body(p0, p1, a0, a1, y1, w_ref, b_ref, y2_ref):
    dis = _dis(p0, p1)
    h = jnp.maximum(dis * (a0[...] + a1[...] + y1[...]) + b_ref[...], 0.0)
    y2_ref[...] = dis * jnp.dot(h, w_ref[...], preferred_element_type=jnp.float32)


def _dense3_body(p0, p1, a0, a1, y2, b_ref, o_ref):
    o_ref[...] = _dis(p0, p1) * (a0[...] + a1[...] + y2[...]) + b_ref[...]


_dense1 = pl.pallas_call(
    _dense1_body, out_shape=jax.ShapeDtypeStruct((N_PAD, D), jnp.float32))
_dense2 = pl.pallas_call(
    _dense2_body, out_shape=jax.ShapeDtypeStruct((N_PAD, D), jnp.float32))
_dense3 = pl.pallas_call(
    _dense3_body, out_shape=jax.ShapeDtypeStruct((N_PAD, D), jnp.float32))


def kernel(x, edge_index, W1, b1, W2, b2):
    ei = edge_index.astype(jnp.int32)
    pad = E_PAD - E
    src = jnp.concatenate(
        [ei[0], jnp.zeros((pad,), jnp.int32)]).reshape(NW, CPW, K)
    dst = jnp.concatenate(
        [ei[1], jnp.full((pad,), N_NODES, jnp.int32)]).reshape(NW, CPW, K)
    x_pad = jnp.zeros((N_PAD, D), jnp.float32).at[:N_NODES].set(x)
    zeros_nd = jnp.zeros((N_PAD, D), jnp.float32)
    zeros_dw = jnp.zeros((N_PAD, DW), jnp.float32)
    ones_k = jnp.ones((K, DW), jnp.float32)

    degp = _deg_sc(dst, ones_k, zeros_dw)
    p0, p1 = degp[0], degp[1]

    y1 = _dense1(p0, p1, x_pad, W1)
    acc1 = _scatter_sc(y1, src, dst, zeros_nd)
    y2 = _dense2(p0, p1, acc1[0], acc1[1], y1, W2, b1.reshape(1, D))
    acc2 = _scatter_sc(y2, src, dst, zeros_nd)
    out = _dense3(p0, p1, acc2[0], acc2[1], y2, b2.reshape(1, D))
    return out[:N_NODES]
